# Initial kernel scaffold; baseline (speedup 1.0000x reference)
import jax
import jax.numpy as jnp
from jax.experimental import pallas as pl


def kernel(x, edge_index, batch, W1_0, b1_0, W2_0, b2_0, gamma_0, beta_0, W1_1, b1_1, W2_1, b2_1, gamma_1, beta_1, W1_2, b1_2, W2_2, b2_2, gamma_2, beta_2):
    raise NotImplementedError("write your pallas kernel here")


# speedup vs baseline: 6.6574x; 6.6574x over previous
import functools

import jax
import jax.numpy as jnp
from jax import lax
from jax.experimental import pallas as pl
from jax.experimental.pallas import tpu as pltpu
from jax.experimental.pallas import tpu_sc as plsc

N = 10000
E = 320000
D = 128
H = 128
G = 64

NC = 2
NS = 16
CHUNK = 80
CHUNKS = E // (NC * NS * CHUNK)
ROWS_PER_TILE = 624
ROWS_TAIL = N - NS * ROWS_PER_TILE

R = 2000
NBLK = N // R

def _sc_segsum_body(h_hbm, edges_hbm, zeros_hbm, out_hbm, src_v, dst_v, rows_v,
                    acc_sh, sem):
    cid = lax.axis_index("c")
    sid = lax.axis_index("s")
    pltpu.sync_copy(edges_hbm.at[0, cid, sid], src_v)
    pltpu.sync_copy(edges_hbm.at[1, cid, sid], dst_v)
    rows0 = sid * ROWS_PER_TILE
    tail0 = NS * ROWS_PER_TILE
    pltpu.sync_copy(zeros_hbm.at[pl.ds(rows0, ROWS_PER_TILE)],
                    acc_sh.at[pl.ds(rows0, ROWS_PER_TILE)])

    @pl.when(sid == NS - 1)
    def _():
        pltpu.sync_copy(zeros_hbm.at[pl.ds(tail0, ROWS_TAIL)],
                        acc_sh.at[pl.ds(tail0, ROWS_TAIL)])

    plsc.subcore_barrier()

    def body(j, carry):
        pltpu.async_copy(h_hbm.at[src_v.at[j]], rows_v, sem).wait()
        pltpu.sync_copy(rows_v, acc_sh.at[dst_v.at[j]], add=True)
        return carry

    lax.fori_loop(0, CHUNKS, body, 0)
    plsc.subcore_barrier()
    pltpu.sync_copy(acc_sh.at[pl.ds(rows0, ROWS_PER_TILE)],
                    out_hbm.at[cid, pl.ds(rows0, ROWS_PER_TILE)])

    @pl.when(sid == NS - 1)
    def _():
        pltpu.sync_copy(acc_sh.at[pl.ds(tail0, ROWS_TAIL)],
                        out_hbm.at[cid, pl.ds(tail0, ROWS_TAIL)])


@functools.cache
def _get_sc_segsum():
    mesh = plsc.VectorSubcoreMesh(core_axis_name="c", subcore_axis_name="s")
    return pl.kernel(
        _sc_segsum_body,
        mesh=mesh,
        out_type=jax.ShapeDtypeStruct((NC, N, D), jnp.float32),
        scratch_types=[
            pltpu.VMEM((CHUNKS, CHUNK), jnp.int32),
            pltpu.VMEM((CHUNKS, CHUNK), jnp.int32),
            pltpu.VMEM((CHUNK, D), jnp.float32),
            pltpu.VMEM_SHARED((N, D), jnp.float32),
            pltpu.SemaphoreType.DMA,
        ],
    )


def _mlp_body(h_ref, p_ref, w1_ref, b1_ref, w2_ref, b2_ref,
              u_ref, s_ref, ss_ref):
    i = pl.program_id(0)
    z = h_ref[...] + p_ref[0] + p_ref[1]
    a = jnp.maximum(
        lax.dot(z, w1_ref[...], preferred_element_type=jnp.float32)
        + b1_ref[...], 0.0)
    u = jnp.maximum(
        lax.dot(a, w2_ref[...], preferred_element_type=jnp.float32)
        + b2_ref[...], 0.0)
    u_ref[...] = u

    @pl.when(i == 0)
    def _():
        s_ref[...] = jnp.zeros_like(s_ref)
        ss_ref[...] = jnp.zeros_like(ss_ref)

    s_ref[...] += jnp.sum(u, axis=0, keepdims=True)
    ss_ref[...] += jnp.sum(u * u, axis=0, keepdims=True)


_tc_mlp = pl.pallas_call(
    _mlp_body,
    grid=(NBLK,),
    in_specs=[
        pl.BlockSpec((R, D), lambda i: (i, 0)),
        pl.BlockSpec((NC, R, D), lambda i: (0, i, 0)),
        pl.BlockSpec((D, H), lambda i: (0, 0)),
        pl.BlockSpec((1, H), lambda i: (0, 0)),
        pl.BlockSpec((H, H), lambda i: (0, 0)),
        pl.BlockSpec((1, H), lambda i: (0, 0)),
    ],
    out_specs=[
        pl.BlockSpec((R, H), lambda i: (i, 0)),
        pl.BlockSpec((1, H), lambda i: (0, 0)),
        pl.BlockSpec((1, H), lambda i: (0, 0)),
    ],
    out_shape=[
        jax.ShapeDtypeStruct((N, H), jnp.float32),
        jax.ShapeDtypeStruct((1, H), jnp.float32),
        jax.ShapeDtypeStruct((1, H), jnp.float32),
    ],
)


def _bn_scale_shift(s, ss, g, b):
    mean = s / N
    var = ss / N - mean * mean
    scale = g * lax.rsqrt(var + 1e-5)
    shift = b - mean * scale
    return scale, shift


def _bn_body(u_ref, s_ref, ss_ref, g_ref, b_ref, h_ref):
    scale, shift = _bn_scale_shift(s_ref[...], ss_ref[...],
                                   g_ref[...], b_ref[...])
    h_ref[...] = u_ref[...] * scale + shift


_tc_bn = pl.pallas_call(
    _bn_body,
    grid=(NBLK,),
    in_specs=[
        pl.BlockSpec((R, H), lambda i: (i, 0)),
        pl.BlockSpec((1, H), lambda i: (0, 0)),
        pl.BlockSpec((1, H), lambda i: (0, 0)),
        pl.BlockSpec((1, H), lambda i: (0, 0)),
        pl.BlockSpec((1, H), lambda i: (0, 0)),
    ],
    out_specs=pl.BlockSpec((R, H), lambda i: (i, 0)),
    out_shape=jax.ShapeDtypeStruct((N, H), jnp.float32),
)


def _bn_pool_body(u_ref, s_ref, ss_ref, g_ref, b_ref, batch_ref,
                  out_ref, acc, cnt):
    i = pl.program_id(0)

    @pl.when(i == 0)
    def _():
        acc[...] = jnp.zeros_like(acc)
        cnt[...] = jnp.zeros_like(cnt)

    scale, shift = _bn_scale_shift(s_ref[...], ss_ref[...],
                                   g_ref[...], b_ref[...])
    hh = u_ref[...] * scale + shift
    b = batch_ref[0, 0, :]
    m = (b[:, None] == lax.broadcasted_iota(jnp.int32, (R, G), 1)
         ).astype(jnp.float32)
    acc[...] += lax.dot_general(m, hh, (((0,), (0,)), ((), ())),
                                preferred_element_type=jnp.float32)
    cnt[...] += lax.dot_general(m, jnp.ones((R, 1), jnp.float32),
                                (((0,), (0,)), ((), ())),
                                preferred_element_type=jnp.float32)

    @pl.when(i == NBLK - 1)
    def _():
        out_ref[...] = acc[...] / jnp.maximum(cnt[...], 1.0)


_tc_bn_pool = pl.pallas_call(
    _bn_pool_body,
    grid=(NBLK,),
    in_specs=[
        pl.BlockSpec((R, H), lambda i: (i, 0)),
        pl.BlockSpec((1, H), lambda i: (0, 0)),
        pl.BlockSpec((1, H), lambda i: (0, 0)),
        pl.BlockSpec((1, H), lambda i: (0, 0)),
        pl.BlockSpec((1, H), lambda i: (0, 0)),
        pl.BlockSpec((1, 1, R), lambda i: (i, 0, 0)),
    ],
    out_specs=pl.BlockSpec((G, H), lambda i: (0, 0)),
    out_shape=jax.ShapeDtypeStruct((G, H), jnp.float32),
    scratch_shapes=[
        pltpu.VMEM((G, H), jnp.float32),
        pltpu.VMEM((G, 1), jnp.float32),
    ],
)


def kernel(x, edge_index, batch, W1_0, b1_0, W2_0, b2_0, gamma_0, beta_0,
           W1_1, b1_1, W2_1, b2_1, gamma_1, beta_1,
           W1_2, b1_2, W2_2, b2_2, gamma_2, beta_2):
    edges_r = edge_index.reshape(2, NC, NS, CHUNKS, CHUNK)
    zeros = jnp.zeros((N, D), jnp.float32)
    batch_r = batch.reshape(NBLK, 1, R)
    params = [
        (W1_0, b1_0, W2_0, b2_0, gamma_0, beta_0),
        (W1_1, b1_1, W2_1, b2_1, gamma_1, beta_1),
        (W1_2, b1_2, W2_2, b2_2, gamma_2, beta_2),
    ]
    h = x
    out = None
    for l in range(3):
        W1, b1, W2, b2, gm, bt = params[l]
        p = _get_sc_segsum()(h, edges_r, zeros)
        u, s, ss = _tc_mlp(h, p, W1, b1.reshape(1, H), W2, b2.reshape(1, H))
        if l < 2:
            h = _tc_bn(u, s, ss, gm.reshape(1, H), bt.reshape(1, H))
        else:
            out = _tc_bn_pool(u, s, ss, gm.reshape(1, H), bt.reshape(1, H),
                              batch_r)
    return out


# speedup vs baseline: 8.4216x; 1.2650x over previous
import functools

import jax
import jax.numpy as jnp
from jax import lax
from jax.experimental import pallas as pl
from jax.experimental.pallas import tpu as pltpu
from jax.experimental.pallas import tpu_sc as plsc

N = 10000
E = 320000
D = 128
H = 128
G = 64

NC = 2
NS = 16
CHUNK = 80
CHUNKS = E // (NC * NS * CHUNK)
EPT = E // (NC * NS)
BUFS = 2
ROUNDS = CHUNKS // BUFS
ROWS_PER_TILE = 624
ROWS_TAIL = N - NS * ROWS_PER_TILE

R = 2000
NBLK = N // R

def _sc_segsum_body(h_hbm, src_hbm, dst_hbm, zeros_hbm, out_hbm,
                    src_v, dst_v, rows_a, rows_b, acc_sh,
                    gsem_a, gsem_b, ssem_a, ssem_b):
    rows = (rows_a, rows_b)
    gsem = (gsem_a, gsem_b)
    ssem = (ssem_a, ssem_b)
    cid = lax.axis_index("c")
    sid = lax.axis_index("s")
    pltpu.sync_copy(src_hbm.at[cid, sid], src_v)
    pltpu.sync_copy(dst_hbm.at[cid, sid], dst_v)
    rows0 = sid * ROWS_PER_TILE
    tail0 = NS * ROWS_PER_TILE
    pltpu.sync_copy(zeros_hbm.at[pl.ds(rows0, ROWS_PER_TILE)],
                    acc_sh.at[pl.ds(rows0, ROWS_PER_TILE)])

    @pl.when(sid == NS - 1)
    def _():
        pltpu.sync_copy(zeros_hbm.at[pl.ds(tail0, ROWS_TAIL)],
                        acc_sh.at[pl.ds(tail0, ROWS_TAIL)])

    plsc.subcore_barrier()

    def _src_slice(j):
        return src_v.at[pl.ds(j * CHUNK, CHUNK)]

    for b in range(BUFS):
        pltpu.async_copy(h_hbm.at[_src_slice(b)], rows[b], gsem[b])

    def body(k, carry):
        j0 = k * BUFS
        for b in range(BUFS):
            pltpu.make_async_copy(h_hbm.at[_src_slice(j0 + b)], rows[b],
                                  gsem[b]).wait()
            pltpu.async_copy(rows[b], acc_sh.at[dst_v.at[j0 + b]],
                             ssem[b], add=True)
        for b in range(BUFS):
            pltpu.make_async_copy(rows[b], acc_sh.at[dst_v.at[j0 + b]],
                                  ssem[b]).wait()

            @pl.when(j0 + BUFS + b < CHUNKS)
            def _():
                pltpu.async_copy(h_hbm.at[_src_slice(j0 + BUFS + b)],
                                 rows[b], gsem[b])

        return carry

    lax.fori_loop(0, ROUNDS, body, 0)
    jlast = ROUNDS * BUFS
    pltpu.make_async_copy(h_hbm.at[_src_slice(jlast)], rows[0],
                          gsem[0]).wait()
    pltpu.sync_copy(rows[0], acc_sh.at[dst_v.at[jlast]], add=True)

    plsc.subcore_barrier()
    pltpu.sync_copy(acc_sh.at[pl.ds(rows0, ROWS_PER_TILE)],
                    out_hbm.at[cid, pl.ds(rows0, ROWS_PER_TILE)])

    @pl.when(sid == NS - 1)
    def _():
        pltpu.sync_copy(acc_sh.at[pl.ds(tail0, ROWS_TAIL)],
                        out_hbm.at[cid, pl.ds(tail0, ROWS_TAIL)])


@functools.cache
def _get_sc_segsum():
    mesh = plsc.VectorSubcoreMesh(core_axis_name="c", subcore_axis_name="s")
    return pl.kernel(
        _sc_segsum_body,
        mesh=mesh,
        out_type=jax.ShapeDtypeStruct((NC, N, D), jnp.float32),
        scratch_types=[
            pltpu.VMEM((EPT,), jnp.int32),
            pltpu.VMEM((CHUNKS, CHUNK), jnp.int32),
            pltpu.VMEM((CHUNK, D), jnp.float32),
            pltpu.VMEM((CHUNK, D), jnp.float32),
            pltpu.VMEM_SHARED((N, D), jnp.float32),
            pltpu.SemaphoreType.DMA,
            pltpu.SemaphoreType.DMA,
            pltpu.SemaphoreType.DMA,
            pltpu.SemaphoreType.DMA,
        ],
    )


def _mlp_body(h_ref, p_ref, w1_ref, b1_ref, w2_ref, b2_ref,
              u_ref, s_ref, ss_ref):
    i = pl.program_id(0)
    z = h_ref[...] + p_ref[0] + p_ref[1]
    a = jnp.maximum(
        lax.dot(z, w1_ref[...], preferred_element_type=jnp.float32)
        + b1_ref[...], 0.0)
    u = jnp.maximum(
        lax.dot(a, w2_ref[...], preferred_element_type=jnp.float32)
        + b2_ref[...], 0.0)
    u_ref[...] = u

    @pl.when(i == 0)
    def _():
        s_ref[...] = jnp.zeros_like(s_ref)
        ss_ref[...] = jnp.zeros_like(ss_ref)

    s_ref[...] += jnp.sum(u, axis=0, keepdims=True)
    ss_ref[...] += jnp.sum(u * u, axis=0, keepdims=True)


_tc_mlp = pl.pallas_call(
    _mlp_body,
    grid=(NBLK,),
    in_specs=[
        pl.BlockSpec((R, D), lambda i: (i, 0)),
        pl.BlockSpec((NC, R, D), lambda i: (0, i, 0)),
        pl.BlockSpec((D, H), lambda i: (0, 0)),
        pl.BlockSpec((1, H), lambda i: (0, 0)),
        pl.BlockSpec((H, H), lambda i: (0, 0)),
        pl.BlockSpec((1, H), lambda i: (0, 0)),
    ],
    out_specs=[
        pl.BlockSpec((R, H), lambda i: (i, 0)),
        pl.BlockSpec((1, H), lambda i: (0, 0)),
        pl.BlockSpec((1, H), lambda i: (0, 0)),
    ],
    out_shape=[
        jax.ShapeDtypeStruct((N, H), jnp.float32),
        jax.ShapeDtypeStruct((1, H), jnp.float32),
        jax.ShapeDtypeStruct((1, H), jnp.float32),
    ],
)


def _bn_scale_shift(s, ss, g, b):
    mean = s / N
    var = ss / N - mean * mean
    scale = g * lax.rsqrt(var + 1e-5)
    shift = b - mean * scale
    return scale, shift


def _bn_body(u_ref, s_ref, ss_ref, g_ref, b_ref, h_ref):
    scale, shift = _bn_scale_shift(s_ref[...], ss_ref[...],
                                   g_ref[...], b_ref[...])
    h_ref[...] = u_ref[...] * scale + shift


_tc_bn = pl.pallas_call(
    _bn_body,
    grid=(NBLK,),
    in_specs=[
        pl.BlockSpec((R, H), lambda i: (i, 0)),
        pl.BlockSpec((1, H), lambda i: (0, 0)),
        pl.BlockSpec((1, H), lambda i: (0, 0)),
        pl.BlockSpec((1, H), lambda i: (0, 0)),
        pl.BlockSpec((1, H), lambda i: (0, 0)),
    ],
    out_specs=pl.BlockSpec((R, H), lambda i: (i, 0)),
    out_shape=jax.ShapeDtypeStruct((N, H), jnp.float32),
)


def _bn_pool_body(u_ref, s_ref, ss_ref, g_ref, b_ref, batch_ref,
                  out_ref, acc, cnt):
    i = pl.program_id(0)

    @pl.when(i == 0)
    def _():
        acc[...] = jnp.zeros_like(acc)
        cnt[...] = jnp.zeros_like(cnt)

    scale, shift = _bn_scale_shift(s_ref[...], ss_ref[...],
                                   g_ref[...], b_ref[...])
    hh = u_ref[...] * scale + shift
    b = batch_ref[0, 0, :]
    m = (b[:, None] == lax.broadcasted_iota(jnp.int32, (R, G), 1)
         ).astype(jnp.float32)
    acc[...] += lax.dot_general(m, hh, (((0,), (0,)), ((), ())),
                                preferred_element_type=jnp.float32)
    cnt[...] += lax.dot_general(m, jnp.ones((R, 1), jnp.float32),
                                (((0,), (0,)), ((), ())),
                                preferred_element_type=jnp.float32)

    @pl.when(i == NBLK - 1)
    def _():
        out_ref[...] = acc[...] / jnp.maximum(cnt[...], 1.0)


_tc_bn_pool = pl.pallas_call(
    _bn_pool_body,
    grid=(NBLK,),
    in_specs=[
        pl.BlockSpec((R, H), lambda i: (i, 0)),
        pl.BlockSpec((1, H), lambda i: (0, 0)),
        pl.BlockSpec((1, H), lambda i: (0, 0)),
        pl.BlockSpec((1, H), lambda i: (0, 0)),
        pl.BlockSpec((1, H), lambda i: (0, 0)),
        pl.BlockSpec((1, 1, R), lambda i: (i, 0, 0)),
    ],
    out_specs=pl.BlockSpec((G, H), lambda i: (0, 0)),
    out_shape=jax.ShapeDtypeStruct((G, H), jnp.float32),
    scratch_shapes=[
        pltpu.VMEM((G, H), jnp.float32),
        pltpu.VMEM((G, 1), jnp.float32),
    ],
)


def kernel(x, edge_index, batch, W1_0, b1_0, W2_0, b2_0, gamma_0, beta_0,
           W1_1, b1_1, W2_1, b2_1, gamma_1, beta_1,
           W1_2, b1_2, W2_2, b2_2, gamma_2, beta_2):
    src_r = edge_index[0].reshape(NC, NS, EPT)
    dst_r = edge_index[1].reshape(NC, NS, CHUNKS, CHUNK)
    zeros = jnp.zeros((N, D), jnp.float32)
    batch_r = batch.reshape(NBLK, 1, R)
    params = [
        (W1_0, b1_0, W2_0, b2_0, gamma_0, beta_0),
        (W1_1, b1_1, W2_1, b2_1, gamma_1, beta_1),
        (W1_2, b1_2, W2_2, b2_2, gamma_2, beta_2),
    ]
    h = x
    out = None
    for l in range(3):
        W1, b1, W2, b2, gm, bt = params[l]
        p = _get_sc_segsum()(h, src_r, dst_r, zeros)
        u, s, ss = _tc_mlp(h, p, W1, b1.reshape(1, H), W2, b2.reshape(1, H))
        if l < 2:
            h = _tc_bn(u, s, ss, gm.reshape(1, H), bt.reshape(1, H))
        else:
            out = _tc_bn_pool(u, s, ss, gm.reshape(1, H), bt.reshape(1, H),
                              batch_r)
    return out
